# baseline (device time: 61138 ns/iter reference)
import jax
import jax.numpy as jnp
from jax import lax
from jax.experimental import pallas as pl
from jax.experimental.pallas import tpu as pltpu

B, S, D = 2, 512, 2048
H, Dh, Dr = 16, 128, 32
DC = 128
BS = B * S
HG = H // 4
CW = HG * Dh
RW = HG * Dr
SCALE = (Dh + Dr) ** -0.5
F32 = jnp.float32
BF16 = jnp.bfloat16

_MESH = pl.DeviceIdType.MESH


def _ring_pos(x, y):
    return 2 * x + (x ^ y)


def _ring_coords(q):
    return (q // 2, (q ^ (q // 2)) & 1)


def _bdot(a, b):
    return lax.dot_general(a, b, (((1,), (0,)), ((), ())),
                           preferred_element_type=F32)


def _body(x_ref, wdkv_ref, wuk_ref, wuv_ref, wq_hbm, wqr_hbm, wkr_ref,
          wo_hbm, out_ref,
          kvown, csend, crecv, wsend, wrecv, wqbuf, wqrbuf,
          q_scr, qr_scr, kr_scr, obuf, wobuf, wo_bf,
          c_sems, w_sems, ring_send_sems, ring_recv_sems, wo_sems, wq_sems):
    my_x = lax.axis_index("x")
    my_y = lax.axis_index("y")
    p = _ring_pos(my_x, my_y)
    pp = _ring_pos(1 - my_x, my_y)
    right = _ring_coords((p + 1) % 4)
    left = _ring_coords((p + 3) % 4)

    wq_fetch = pltpu.make_async_copy(
        wq_hbm.at[:, pl.ds(p * CW, CW)], wqbuf, wq_sems.at[0])
    wqr_fetch = pltpu.make_async_copy(
        wqr_hbm.at[:, pl.ds(p * RW, RW)], wqrbuf, wq_sems.at[1])
    wq_fetch.start()
    wqr_fetch.start()
    origin0 = p
    origin1 = (p + 3) % 4
    wo_fetch0 = pltpu.make_async_copy(
        wo_hbm.at[pl.ds(origin0 * CW, CW), :], wobuf.at[0], wo_sems.at[0])
    wo_fetch1 = pltpu.make_async_copy(
        wo_hbm.at[pl.ds(origin1 * CW, CW), :], wobuf.at[1], wo_sems.at[1])
    wo_fetch0.start()
    wo_fetch1.start()

    x_bf = x_ref[...].astype(BF16)
    csend[...] = _bdot(x_bf, wdkv_ref[...].astype(BF16)).astype(BF16)
    wsend[0] = wuk_ref[:, pl.ds(pp * CW, CW)].astype(BF16)
    wsend[1] = wuv_ref[:, pl.ds(pp * CW, CW)].astype(BF16)

    barrier = pltpu.get_barrier_semaphore()
    for k in range(1, 4):
        pl.semaphore_signal(barrier, inc=1,
                            device_id=_ring_coords((p + k) % 4),
                            device_id_type=_MESH)
    pl.semaphore_wait(barrier, 3)

    w_rdma = pltpu.make_async_remote_copy(
        src_ref=wsend, dst_ref=wrecv,
        send_sem=w_sems.at[0], recv_sem=w_sems.at[1],
        device_id=(1 - my_x, my_y), device_id_type=_MESH)
    w_rdma.start()
    c_rdmas = []
    for b in range(B):
        rows = pl.ds(b * S, S)
        r = pltpu.make_async_remote_copy(
            src_ref=csend.at[rows, :], dst_ref=crecv.at[rows, :],
            send_sem=c_sems.at[2 * b], recv_sem=c_sems.at[2 * b + 1],
            device_id=(1 - my_x, my_y), device_id_type=_MESH)
        r.start()
        c_rdmas.append(r)

    wq_fetch.wait()
    wqr_fetch.wait()
    wq_bf = wqbuf[...].astype(BF16)
    wqr_bf = wqrbuf[...].astype(BF16)
    wkr_bf = wkr_ref[...].astype(BF16)
    wuk_own = wuk_ref[:, pl.ds(p * CW, CW)].astype(BF16)
    wuv_own = wuv_ref[:, pl.ds(p * CW, CW)].astype(BF16)
    dims_t = (((1,), (1,)), ((), ()))
    ring = [[None, None], [None, None], [None, None]]

    def _send(half):
        rows = pl.ds(half * S, S)
        for k in range(1, 4):
            idx = 2 * (k - 1) + half
            r = pltpu.make_async_remote_copy(
                src_ref=obuf.at[0, rows, :], dst_ref=obuf.at[k, rows, :],
                send_sem=ring_send_sems.at[idx],
                recv_sem=ring_recv_sems.at[idx],
                device_id=_ring_coords((p + k) % 4), device_id_type=_MESH)
            r.start()
            ring[k - 1][half] = r

    for b in range(B):
        rows = slice(b * S, (b + 1) * S)
        x_b = x_bf[rows, :]
        q_scr[rows, :] = _bdot(x_b, wq_bf).astype(BF16)
        qr_scr[rows, :] = _bdot(x_b, wqr_bf).astype(BF16)
        kr_scr[rows, :] = _bdot(x_b, wkr_bf).astype(BF16)
        k_part = _bdot(csend[rows, :], wuk_own)
        v_part = _bdot(csend[rows, :], wuv_own)
        if b == 0:
            w_rdma.wait()
        c_rdmas[b].wait()
        kvown[0, rows, :] = (k_part + _bdot(crecv[rows, :], wrecv[0])
                             ).astype(BF16)
        kvown[1, rows, :] = (v_part + _bdot(crecv[rows, :], wrecv[1])
                             ).astype(BF16)

        kr_b = kr_scr[rows, :]
        qr_b = qr_scr[rows, :]
        for j in range(HG):
            cols = slice(j * Dh, (j + 1) * Dh)
            q_bh = q_scr[rows, cols]
            k_bh = kvown[0, rows, cols]
            qr_bh = qr_b[:, j * Dr:(j + 1) * Dr]
            s = (lax.dot_general(q_bh, k_bh, dims_t,
                                 preferred_element_type=F32)
                 + lax.dot_general(qr_bh, kr_b, dims_t,
                                   preferred_element_type=F32)) * SCALE
            e = jnp.exp(s)
            denom = jnp.sum(e, axis=-1, keepdims=True)
            o = _bdot(e.astype(BF16), kvown[1, rows, cols]) / denom
            obuf[0, rows, cols] = o.astype(BF16)
        _send(b)

    wo_fetch0.wait()
    wo_bf[0] = wobuf[0].astype(BF16)
    out_ref[...] = _bdot(obuf[0], wo_bf[0])
    origin2 = (p + 2) % 4
    wo_fetch2 = pltpu.make_async_copy(
        wo_hbm.at[pl.ds(origin2 * CW, CW), :], wobuf.at[0], wo_sems.at[0])
    wo_fetch2.start()
    wo_fetch1.wait()
    wo_bf[1] = wobuf[1].astype(BF16)

    for h in range(1, 3):
        for half in range(B):
            ring[h - 1][half].wait()
            rows = slice(half * S, (half + 1) * S)
            out_ref[rows, :] = out_ref[rows, :] + _bdot(
                obuf[h, rows, :], wo_bf[h % 2])
        if h == 1:
            origin3 = (p + 1) % 4
            wo_fetch3 = pltpu.make_async_copy(
                wo_hbm.at[pl.ds(origin3 * CW, CW), :], wobuf.at[1],
                wo_sems.at[1])
            wo_fetch3.start()
            wo_fetch2.wait()
            wo_bf[0] = wobuf[0].astype(BF16)

    wo_fetch3.wait()
    wo_bf[1] = wobuf[1].astype(BF16)
    for half in range(B):
        ring[2][half].wait()
        rows = slice(half * S, (half + 1) * S)
        out_ref[rows, :] = out_ref[rows, :] + _bdot(
            obuf[3, rows, :], wo_bf[1])


def kernel(x, Wdkv, Wuk, Wuv, Wq, Wqr, Wkr, Wo):
    x2d = x.reshape(BS, D)

    out2d = pl.pallas_call(
        _body,
        out_shape=jax.ShapeDtypeStruct((BS, D), F32),
        in_specs=[pl.BlockSpec(memory_space=pltpu.VMEM)] * 4
        + [pl.BlockSpec(memory_space=pl.ANY)] * 2
        + [pl.BlockSpec(memory_space=pltpu.VMEM)]
        + [pl.BlockSpec(memory_space=pl.ANY)],
        out_specs=pl.BlockSpec(memory_space=pltpu.VMEM),
        scratch_shapes=[
            pltpu.VMEM((2, BS, CW), BF16),
            pltpu.VMEM((BS, DC), BF16),
            pltpu.VMEM((BS, DC), BF16),
            pltpu.VMEM((2, DC, CW), BF16),
            pltpu.VMEM((2, DC, CW), BF16),
            pltpu.VMEM((D, CW), F32),
            pltpu.VMEM((D, RW), F32),
            pltpu.VMEM((BS, CW), BF16),
            pltpu.VMEM((BS, RW), BF16),
            pltpu.VMEM((BS, Dr), BF16),
            pltpu.VMEM((4, BS, CW), BF16),
            pltpu.VMEM((2, CW, D), F32),
            pltpu.VMEM((2, CW, D), BF16),
            pltpu.SemaphoreType.DMA((4,)),
            pltpu.SemaphoreType.DMA((2,)),
            pltpu.SemaphoreType.DMA((6,)),
            pltpu.SemaphoreType.DMA((6,)),
            pltpu.SemaphoreType.DMA((2,)),
            pltpu.SemaphoreType.DMA((2,)),
        ],
        compiler_params=pltpu.CompilerParams(
            collective_id=0, vmem_limit_bytes=60 * 1024 * 1024),
    )(x2d, Wdkv, Wuk, Wuv, Wq, Wqr, Wkr, Wo)
    return out2d.reshape(B, S, D)


# device time: 60551 ns/iter; 1.0097x vs baseline; 1.0097x over previous
import jax
import jax.numpy as jnp
from jax import lax
from jax.experimental import pallas as pl
from jax.experimental.pallas import tpu as pltpu

B, S, D = 2, 512, 2048
H, Dh, Dr = 16, 128, 32
DC = 128
BS = B * S
HG = H // 4
CW = HG * Dh
RW = HG * Dr
SCALE = (Dh + Dr) ** -0.5
F32 = jnp.float32
BF16 = jnp.bfloat16

_MESH = pl.DeviceIdType.MESH


def _ring_pos(x, y):
    return 2 * x + (x ^ y)


def _ring_coords(q):
    return (q // 2, (q ^ (q // 2)) & 1)


def _bdot(a, b):
    return lax.dot_general(a, b, (((1,), (0,)), ((), ())),
                           preferred_element_type=F32)


def _body(x_ref, wdkv_ref, wuk_ref, wuv_ref, wq_hbm, wqr_hbm, wkr_ref,
          wo_hbm, out_ref,
          kvown, csend, crecv, wsend, wrecv, wqbuf, wqrbuf,
          q_scr, qr_scr, kr_scr, obuf, wobuf,
          c_sems, w_sems, ring_send_sems, ring_recv_sems, wo_sems, wq_sems):
    my_x = lax.axis_index("x")
    my_y = lax.axis_index("y")
    p = _ring_pos(my_x, my_y)
    pp = _ring_pos(1 - my_x, my_y)
    right = _ring_coords((p + 1) % 4)
    left = _ring_coords((p + 3) % 4)

    wq_fetch = pltpu.make_async_copy(
        wq_hbm.at[:, pl.ds(p * CW, CW)], wqbuf, wq_sems.at[0])
    wqr_fetch = pltpu.make_async_copy(
        wqr_hbm.at[:, pl.ds(p * RW, RW)], wqrbuf, wq_sems.at[1])
    wq_fetch.start()
    wqr_fetch.start()
    origin0 = p
    origin1 = (p + 3) % 4
    wo_fetch0 = pltpu.make_async_copy(
        wo_hbm.at[pl.ds(origin0 * CW, CW), :], wobuf.at[0], wo_sems.at[0])
    wo_fetch1 = pltpu.make_async_copy(
        wo_hbm.at[pl.ds(origin1 * CW, CW), :], wobuf.at[1], wo_sems.at[1])
    wo_fetch0.start()
    wo_fetch1.start()

    x_bf = x_ref[...].astype(BF16)
    csend[...] = _bdot(x_bf, wdkv_ref[...].astype(BF16)).astype(BF16)
    wsend[0] = wuk_ref[:, pl.ds(pp * CW, CW)].astype(BF16)
    wsend[1] = wuv_ref[:, pl.ds(pp * CW, CW)].astype(BF16)

    barrier = pltpu.get_barrier_semaphore()
    for k in range(1, 4):
        pl.semaphore_signal(barrier, inc=1,
                            device_id=_ring_coords((p + k) % 4),
                            device_id_type=_MESH)
    pl.semaphore_wait(barrier, 3)

    w_rdma = pltpu.make_async_remote_copy(
        src_ref=wsend, dst_ref=wrecv,
        send_sem=w_sems.at[0], recv_sem=w_sems.at[1],
        device_id=(1 - my_x, my_y), device_id_type=_MESH)
    w_rdma.start()
    c_rdmas = []
    for b in range(B):
        rows = pl.ds(b * S, S)
        r = pltpu.make_async_remote_copy(
            src_ref=csend.at[rows, :], dst_ref=crecv.at[rows, :],
            send_sem=c_sems.at[2 * b], recv_sem=c_sems.at[2 * b + 1],
            device_id=(1 - my_x, my_y), device_id_type=_MESH)
        r.start()
        c_rdmas.append(r)

    wq_fetch.wait()
    wqr_fetch.wait()
    wq_bf = wqbuf[...].astype(BF16)
    wqr_bf = wqrbuf[...].astype(BF16)
    wkr_bf = wkr_ref[...].astype(BF16)
    wuk_own = wuk_ref[:, pl.ds(p * CW, CW)].astype(BF16)
    wuv_own = wuv_ref[:, pl.ds(p * CW, CW)].astype(BF16)
    dims_t = (((1,), (1,)), ((), ()))
    ring = [[None, None], [None, None], [None, None]]

    def _send(half):
        rows = pl.ds(half * S, S)
        for k in range(1, 4):
            idx = 2 * (k - 1) + half
            r = pltpu.make_async_remote_copy(
                src_ref=obuf.at[0, rows, :], dst_ref=obuf.at[k, rows, :],
                send_sem=ring_send_sems.at[idx],
                recv_sem=ring_recv_sems.at[idx],
                device_id=_ring_coords((p + k) % 4), device_id_type=_MESH)
            r.start()
            ring[k - 1][half] = r

    for b in range(B):
        rows = slice(b * S, (b + 1) * S)
        x_b = x_bf[rows, :]
        q_scr[rows, :] = _bdot(x_b, wq_bf).astype(BF16)
        qr_scr[rows, :] = _bdot(x_b, wqr_bf).astype(BF16)
        kr_scr[rows, :] = _bdot(x_b, wkr_bf).astype(BF16)
        k_part = _bdot(csend[rows, :], wuk_own)
        v_part = _bdot(csend[rows, :], wuv_own)
        if b == 0:
            w_rdma.wait()
        c_rdmas[b].wait()
        kvown[0, rows, :] = (k_part + _bdot(crecv[rows, :], wrecv[0])
                             ).astype(BF16)
        kvown[1, rows, :] = (v_part + _bdot(crecv[rows, :], wrecv[1])
                             ).astype(BF16)

        kr_b = kr_scr[rows, :]
        qr_b = qr_scr[rows, :]
        for j in range(HG):
            cols = slice(j * Dh, (j + 1) * Dh)
            q_bh = q_scr[rows, cols]
            k_bh = kvown[0, rows, cols]
            qr_bh = qr_b[:, j * Dr:(j + 1) * Dr]
            s = (lax.dot_general(q_bh, k_bh, dims_t,
                                 preferred_element_type=F32)
                 + lax.dot_general(qr_bh, kr_b, dims_t,
                                   preferred_element_type=F32)) * SCALE
            e = jnp.exp(s)
            denom = jnp.sum(e, axis=-1, keepdims=True)
            o = _bdot(e.astype(BF16), kvown[1, rows, cols]) / denom
            obuf[0, rows, cols] = o.astype(BF16)
        _send(b)

    wo_fetch0.wait()
    out_ref[...] = _bdot(obuf[0], wobuf[0].astype(BF16))
    origin2 = (p + 2) % 4
    wo_fetch2 = pltpu.make_async_copy(
        wo_hbm.at[pl.ds(origin2 * CW, CW), :], wobuf.at[0], wo_sems.at[0])
    wo_fetch2.start()

    for h in range(1, 3):
        wo_fetchN = wo_fetch1 if h == 1 else wo_fetch2
        for half in range(B):
            ring[h - 1][half].wait()
            if half == 0:
                wo_fetchN.wait()
            rows = slice(half * S, (half + 1) * S)
            out_ref[rows, :] = out_ref[rows, :] + _bdot(
                obuf[h, rows, :], wobuf[h % 2].astype(BF16))
        if h == 1:
            origin3 = (p + 1) % 4
            wo_fetch3 = pltpu.make_async_copy(
                wo_hbm.at[pl.ds(origin3 * CW, CW), :], wobuf.at[1],
                wo_sems.at[1])
            wo_fetch3.start()

    for half in range(B):
        ring[2][half].wait()
        if half == 0:
            wo_fetch3.wait()
        rows = slice(half * S, (half + 1) * S)
        out_ref[rows, :] = out_ref[rows, :] + _bdot(
            obuf[3, rows, :], wobuf[1].astype(BF16))


def kernel(x, Wdkv, Wuk, Wuv, Wq, Wqr, Wkr, Wo):
    x2d = x.reshape(BS, D)

    out2d = pl.pallas_call(
        _body,
        out_shape=jax.ShapeDtypeStruct((BS, D), F32),
        in_specs=[pl.BlockSpec(memory_space=pltpu.VMEM)] * 4
        + [pl.BlockSpec(memory_space=pl.ANY)] * 2
        + [pl.BlockSpec(memory_space=pltpu.VMEM)]
        + [pl.BlockSpec(memory_space=pl.ANY)],
        out_specs=pl.BlockSpec(memory_space=pltpu.VMEM),
        scratch_shapes=[
            pltpu.VMEM((2, BS, CW), BF16),
            pltpu.VMEM((BS, DC), BF16),
            pltpu.VMEM((BS, DC), BF16),
            pltpu.VMEM((2, DC, CW), BF16),
            pltpu.VMEM((2, DC, CW), BF16),
            pltpu.VMEM((D, CW), F32),
            pltpu.VMEM((D, RW), F32),
            pltpu.VMEM((BS, CW), BF16),
            pltpu.VMEM((BS, RW), BF16),
            pltpu.VMEM((BS, Dr), BF16),
            pltpu.VMEM((4, BS, CW), BF16),
            pltpu.VMEM((2, CW, D), F32),
            pltpu.SemaphoreType.DMA((4,)),
            pltpu.SemaphoreType.DMA((2,)),
            pltpu.SemaphoreType.DMA((6,)),
            pltpu.SemaphoreType.DMA((6,)),
            pltpu.SemaphoreType.DMA((2,)),
            pltpu.SemaphoreType.DMA((2,)),
        ],
        compiler_params=pltpu.CompilerParams(
            collective_id=0, vmem_limit_bytes=60 * 1024 * 1024),
    )(x2d, Wdkv, Wuk, Wuv, Wq, Wqr, Wkr, Wo)
    return out2d.reshape(B, S, D)
